# baseline (speedup 1.0000x reference)
import functools
import math

import jax
import jax.numpy as jnp
from jax import lax
from jax.experimental import pallas as pl
from jax.experimental.pallas import tpu as pltpu
from jax.experimental.pallas import tpu_sc as plsc

D_MODEL = 1024
SCALE = math.sqrt(D_MODEL)
EPS = 1e-5
LANES = 16
NUM_CORES = 2
NUM_SUBCORES = 16
NUM_WORKERS = NUM_CORES * NUM_SUBCORES
CHUNK_ROWS = 16
GROUPS = D_MODEL // LANES


def _rsqrt_newton(v):
    i = lax.bitcast_convert_type(v, jnp.int32)
    i = jnp.int32(0x5F3759DF) - lax.shift_right_logical(i, 1)
    y = lax.bitcast_convert_type(i, jnp.float32)
    for _ in range(3):
        y = y * (1.5 - 0.5 * v * y * y)
    return y


def _make_sc_kernel(n_rows):
    rows_per_w = n_rows // NUM_WORKERS
    n_chunks = rows_per_w // CHUNK_ROWS
    assert n_chunks % 2 == 0

    mesh = plsc.VectorSubcoreMesh(core_axis_name="c", subcore_axis_name="s")

    @functools.partial(
        pl.kernel,
        out_type=jax.ShapeDtypeStruct((n_rows, D_MODEL), jnp.float32),
        mesh=mesh,
        compiler_params=pltpu.CompilerParams(needs_layout_passes=False),
        scratch_types=[
            pltpu.VMEM((2, CHUNK_ROWS), jnp.int32),
            pltpu.VMEM((2, CHUNK_ROWS, D_MODEL), jnp.float32),
            pltpu.VMEM((2, CHUNK_ROWS, D_MODEL), jnp.float32),
            pltpu.VMEM((D_MODEL,), jnp.float32),
            pltpu.VMEM((D_MODEL,), jnp.float32),
            pltpu.SMEM((CHUNK_ROWS,), jnp.float32),
            pltpu.SMEM((CHUNK_ROWS,), jnp.float32),
            pltpu.SemaphoreType.DMA,
            pltpu.SemaphoreType.DMA,
            pltpu.SemaphoreType.DMA,
            pltpu.SemaphoreType.DMA,
            pltpu.SemaphoreType.DMA,
            pltpu.SemaphoreType.DMA,
        ],
    )
    def sc_kernel(x_hbm, pos_hbm, pe_hbm, w_hbm, b_hbm, out_hbm,
                  idx2, pe2, x2, w_v, b_v, mean_a, rstd_a,
                  gsem0, gsem1, xsem0, xsem1, osem0, osem1):
        gsems = (gsem0, gsem1)
        xsems = (xsem0, xsem1)
        osems = (osem0, osem1)
        wid = lax.axis_index("s") * NUM_CORES + lax.axis_index("c")
        base_w = wid * rows_per_w

        pltpu.sync_copy(w_hbm, w_v)
        pltpu.sync_copy(b_hbm, b_v)

        def stage(c, p):
            base = base_w + c * CHUNK_ROWS
            pltpu.sync_copy(pos_hbm.at[pl.ds(base, CHUNK_ROWS)], idx2.at[p])
            pltpu.make_async_copy(
                pe_hbm.at[idx2.at[p]], pe2.at[p], gsems[p]).start()
            pltpu.make_async_copy(
                x_hbm.at[pl.ds(base, CHUNK_ROWS)], x2.at[p], xsems[p]).start()

        def wait_in(c, p):
            pltpu.make_async_copy(
                pe_hbm.at[idx2.at[p]], pe2.at[p], gsems[p]).wait()
            base = base_w + c * CHUNK_ROWS
            pltpu.make_async_copy(
                x_hbm.at[pl.ds(base, CHUNK_ROWS)], x2.at[p], xsems[p]).wait()

        def start_out(c, p):
            base = base_w + c * CHUNK_ROWS
            pltpu.make_async_copy(
                x2.at[p], out_hbm.at[pl.ds(base, CHUNK_ROWS)], osems[p]).start()

        def wait_out(c, p):
            base = base_w + c * CHUNK_ROWS
            pltpu.make_async_copy(
                x2.at[p], out_hbm.at[pl.ds(base, CHUNK_ROWS)], osems[p]).wait()

        def compute(p):
            xb = x2.at[p]
            pb = pe2.at[p]

            def phase_a(r, _):
                nacc = 4
                zero = jnp.zeros((LANES,), jnp.float32)
                s_acc = [zero] * nacc
                q_acc = [zero] * nacc
                for j in range(GROUPS):
                    v = xb[r, pl.ds(j * LANES, LANES)]
                    k = j % nacc
                    s_acc[k] = s_acc[k] + v
                    q_acc[k] = q_acc[k] + v * v
                s = (s_acc[0] + s_acc[1]) + (s_acc[2] + s_acc[3])
                sq = (q_acc[0] + q_acc[1]) + (q_acc[2] + q_acc[3])
                inv_d = jnp.float32(1.0 / D_MODEL)
                mean = jnp.sum(s) * inv_d
                var = jnp.sum(sq) * inv_d - mean * mean
                mean_a[r] = mean
                rstd_a[r] = _rsqrt_newton(var + EPS)
                return 0

            lax.fori_loop(0, CHUNK_ROWS, phase_a, 0)

            def phase_b(j, _):
                sl = pl.ds(j * LANES, LANES)
                wv = w_v[sl]
                bv = b_v[sl]
                for r in range(CHUNK_ROWS):
                    mv = jnp.full((LANES,), mean_a[r], jnp.float32)
                    rv = jnp.full((LANES,), rstd_a[r], jnp.float32)
                    xv = xb[r, sl]
                    pv = pb[r, sl]
                    xb[r, sl] = (xv - mv) * rv * wv + (bv + pv * SCALE)
                return 0

            lax.fori_loop(0, GROUPS, phase_b, 0)

        stage(0, 0)

        def pair_body(cc, _):
            for p in range(2):
                c = cc * 2 + p
                q = 1 - p

                @pl.when(c + 1 < n_chunks)
                def _():
                    @pl.when(c >= 1)
                    def _():
                        wait_out(c - 1, q)

                    stage(c + 1, q)

                wait_in(c, p)
                compute(p)
                start_out(c, p)
            return 0

        lax.fori_loop(0, n_chunks // 2, pair_body, 0)
        wait_out(n_chunks - 2, 0)
        wait_out(n_chunks - 1, 1)

    return sc_kernel


def kernel(x, positions, pe, ln_w, ln_b):
    b, t, d = x.shape
    n = b * t
    xf = x.reshape(n, d)
    posf = positions.reshape(n).astype(jnp.int32)
    out = _make_sc_kernel(n)(xf, posf, pe, ln_w, ln_b)
    return out.reshape(b, t, d)
